# baseline (speedup 1.0000x reference)
import functools

import jax
import jax.numpy as jnp
from jax import lax
from jax.experimental import pallas as pl
from jax.experimental.pallas import tpu as pltpu
from jax.experimental.pallas import tpu_sc as plsc

D_MODEL = 64
NUM_CORES = 2
NUM_SUBCORES = 16
NUM_WORKERS = NUM_CORES * NUM_SUBCORES

CHUNK_ROWS = 1024
GATHERS_PER_CHUNK = CHUNK_ROWS // 128


def _make_lookup(total_rows: int):
  assert total_rows % (NUM_WORKERS * CHUNK_ROWS) == 0
  rows_per_worker = total_rows // NUM_WORKERS
  chunks_per_worker = rows_per_worker // CHUNK_ROWS

  mesh = plsc.VectorSubcoreMesh(
      core_axis_name="c", subcore_axis_name="s")

  @functools.partial(
      pl.kernel,
      out_type=jax.ShapeDtypeStruct((total_rows, D_MODEL), jnp.float32),
      mesh=mesh,
      scratch_types=[
          pltpu.VMEM((GATHERS_PER_CHUNK, 128), jnp.int32),
          pltpu.VMEM((CHUNK_ROWS, D_MODEL), jnp.float32),
          pltpu.SemaphoreType.DMA,
      ],
      compiler_params=pltpu.CompilerParams(use_tc_tiling_on_sc=False),
  )
  def lookup(table_hbm, idx_hbm, out_hbm, idx_v, rows_v, sem):
    wid = lax.axis_index("s") * NUM_CORES + lax.axis_index("c")
    base = wid * rows_per_worker

    def chunk_body(c, carry):
      pltpu.sync_copy(idx_hbm.at[wid, c], idx_v)
      copies = []
      for j in range(GATHERS_PER_CHUNK):
        copies.append(
            pltpu.async_copy(
                table_hbm.at[idx_v.at[j]],
                rows_v.at[pl.ds(j * 128, 128)],
                sem,
            ))
      for cp in copies:
        cp.wait()
      pltpu.sync_copy(
          rows_v, out_hbm.at[pl.ds(base + c * CHUNK_ROWS, CHUNK_ROWS)])
      return carry

    lax.fori_loop(0, chunks_per_worker, chunk_body, 0, unroll=False)

  return lookup


@jax.jit
def kernel(x, table):
  b, t = x.shape
  total = b * t
  idx = x.reshape(
      NUM_WORKERS, total // (NUM_WORKERS * CHUNK_ROWS), GATHERS_PER_CHUNK, 128
  ).astype(jnp.int32)
  out = _make_lookup(total)(table, idx)
  return out.reshape(b, t, D_MODEL)


# speedup vs baseline: 1.0293x; 1.0293x over previous
import functools

import jax
import jax.numpy as jnp
from jax import lax
from jax.experimental import pallas as pl
from jax.experimental.pallas import tpu as pltpu
from jax.experimental.pallas import tpu_sc as plsc

D_MODEL = 64
NUM_CORES = 2
NUM_SUBCORES = 16
NUM_WORKERS = NUM_CORES * NUM_SUBCORES

CHUNK_ROWS = 512
GPC = CHUNK_ROWS // 128
N_ROWBUF = 2
N_IDXBUF = 4


def _make_lookup(total_rows: int):
  assert total_rows % (NUM_WORKERS * CHUNK_ROWS) == 0
  rows_per_worker = total_rows // NUM_WORKERS
  n_chunks = rows_per_worker // CHUNK_ROWS
  assert (n_chunks - 4) % 4 == 0 and n_chunks >= 8

  mesh = plsc.VectorSubcoreMesh(
      core_axis_name="c", subcore_axis_name="s")

  @functools.partial(
      pl.kernel,
      out_type=jax.ShapeDtypeStruct((total_rows, D_MODEL), jnp.float32),
      mesh=mesh,
      scratch_types=(
          [pltpu.VMEM((GPC, 128), jnp.int32) for _ in range(N_IDXBUF)]
          + [pltpu.VMEM((CHUNK_ROWS, D_MODEL), jnp.float32)
             for _ in range(N_ROWBUF)]
          + [pltpu.SemaphoreType.DMA] * (N_IDXBUF + 2 * N_ROWBUF)
      ),
      compiler_params=pltpu.CompilerParams(use_tc_tiling_on_sc=False),
  )
  def lookup(table_hbm, idx_hbm, out_hbm, *bufs):
    idx_v = bufs[:N_IDXBUF]
    rows_v = bufs[N_IDXBUF:N_IDXBUF + N_ROWBUF]
    sem_i = bufs[N_IDXBUF + N_ROWBUF:2 * N_IDXBUF + N_ROWBUF]
    sem_g = bufs[2 * N_IDXBUF + N_ROWBUF:2 * N_IDXBUF + 2 * N_ROWBUF]
    sem_o = bufs[2 * N_IDXBUF + 2 * N_ROWBUF:]

    wid = lax.axis_index("s") * NUM_CORES + lax.axis_index("c")
    base = wid * rows_per_worker

    def fire_idx(c, q):
      pltpu.async_copy(idx_hbm.at[wid, c], idx_v[q], sem_i[q])

    def wait_idx(c, q):
      pltpu.make_async_copy(idx_hbm.at[wid, c], idx_v[q], sem_i[q]).wait()

    def fire_gathers(q, p):
      for j in range(GPC):
        pltpu.async_copy(
            table_hbm.at[idx_v[q].at[j]],
            rows_v[p].at[pl.ds(j * 128, 128)],
            sem_g[p],
        )

    def wait_gathers(q, p):
      for j in range(GPC):
        pltpu.make_async_copy(
            table_hbm.at[idx_v[q].at[j]],
            rows_v[p].at[pl.ds(j * 128, 128)],
            sem_g[p],
        ).wait()

    def out_ref_for(c):
      return out_hbm.at[pl.ds(base + c * CHUNK_ROWS, CHUNK_ROWS)]

    def fire_out(c, p):
      pltpu.async_copy(rows_v[p], out_ref_for(c), sem_o[p])

    def wait_out(c, p):
      pltpu.make_async_copy(rows_v[p], out_ref_for(c), sem_o[p]).wait()

    for c0 in range(N_IDXBUF):
      fire_idx(c0, c0)
    wait_idx(0, 0)
    fire_gathers(0, 0)
    wait_idx(1, 1)
    fire_gathers(1, 1)
    wait_gathers(0, 0)
    fire_out(0, 0)

    def body(g, carry):
      for b in range(4):
        c = 4 * g + 2 + b
        p = (2 + b) % 2
        q = (2 + b) % N_IDXBUF
        wait_out(c - 2, p)
        wait_idx(c, q)
        fire_gathers(q, p)
        wait_gathers((q - 1) % N_IDXBUF, 1 - p)
        fire_out(c - 1, 1 - p)
        fire_idx(c + 2, (q + 2) % N_IDXBUF)
      return carry

    lax.fori_loop(0, (n_chunks - 4) // 4, body, 0, unroll=False)

    for c in (n_chunks - 2, n_chunks - 1):
      p = c % 2
      q = c % N_IDXBUF
      wait_out(c - 2, p)
      wait_idx(c, q)
      fire_gathers(q, p)
      wait_gathers((q - 1) % N_IDXBUF, 1 - p)
      fire_out(c - 1, 1 - p)
    pl_last = (n_chunks - 1) % 2
    wait_gathers((n_chunks - 1) % N_IDXBUF, pl_last)
    fire_out(n_chunks - 1, pl_last)
    wait_out(n_chunks - 2, 1 - pl_last)
    wait_out(n_chunks - 1, pl_last)

  return lookup


@jax.jit
def kernel(x, table):
  b, t = x.shape
  total = b * t
  idx = x.reshape(
      NUM_WORKERS, total // (NUM_WORKERS * CHUNK_ROWS), GPC, 128
  ).astype(jnp.int32)
  out = _make_lookup(total)(table, idx)
  return out.reshape(b, t, D_MODEL)
